# baseline (speedup 1.0000x reference)
import functools

import jax
import jax.numpy as jnp
from jax import lax
from jax.experimental import pallas as pl
from jax.experimental.pallas import tpu as pltpu
from jax.experimental.pallas import tpu_sc as plsc

H = 32
NE = 1536
NS = 512
MHD = 5
NSLAB = MHD * 3

_NC = 2
_NSUB = 16
_NW = _NC * _NSUB

_CHUNK = 64


def _tmul_body(ew_ref, w_ref, out_ref):
    out_ref[0] = jnp.dot(ew_ref[...], w_ref[0],
                         preferred_element_type=jnp.float32) * (1.0 / 3.0)


def _build_table(edge_enc_w, w5):
    return pl.pallas_call(
        _tmul_body,
        grid=(MHD,),
        in_specs=[
            pl.BlockSpec((NE + 1, H), lambda d: (0, 0)),
            pl.BlockSpec((1, H, H), lambda d: (d, 0, 0)),
        ],
        out_specs=pl.BlockSpec((1, NE + 1, H), lambda d: (d, 0, 0)),
        out_shape=jax.ShapeDtypeStruct((MHD, NE + 1, H), jnp.float32),
    )(edge_enc_w, w5)


def _sc_body(nchunks, pn, t0, t1, t2, t3, t4, cidx, spos, spw, eo_lin, spb_lin,
             idx0, idx1, rows0, rows1, acct0, acct1, sidx0, sidx1,
             srow0, srow1, spbt0, spbt1, r_v,
             gsem0, gsem1, wsem0, wsem1):
    tbls = (t0, t1, t2, t3, t4)
    w = lax.axis_index("s") * _NC + lax.axis_index("c")
    base = w * pn
    lanes = jnp.arange(16, dtype=jnp.int32)
    bufs = ((idx0, rows0, acct0, sidx0, srow0, spbt0, gsem0, wsem0),
            (idx1, rows1, acct1, sidx1, srow1, spbt1, gsem1, wsem1))

    def stage_and_fire(g, bq):
        idx_q, rows_q, _, sidx_q, srow_q, _, gsem_q, _ = bq
        pij0 = pl.multiple_of(g * _CHUNK, _CHUNK)
        pos0 = pl.multiple_of(base + pij0, _CHUNK)
        pltpu.sync_copy(cidx.at[w, :, pl.ds(pij0, _CHUNK)], idx_q)
        pltpu.sync_copy(spos.at[pl.ds(pos0, _CHUNK)], sidx_q)
        for m in range(NSLAB):
            pltpu.async_copy(tbls[m // 3].at[idx_q.at[m]],
                             rows_q.at[pl.ds(m * _CHUNK, _CHUNK)], gsem_q)
        pltpu.async_copy(spw.at[sidx_q], srow_q, gsem_q)

    def drain_gathers(bp):
        _, rows_p, _, _, srow_p, _, gsem_p, _ = bp
        pltpu.make_async_copy(t0.at[pl.ds(0, NSLAB * _CHUNK)], rows_p,
                              gsem_p).wait()
        pltpu.make_async_copy(t0.at[pl.ds(0, _CHUNK)], srow_p, gsem_p).wait()

    def drain_writes(bq):
        _, _, acct_q, _, _, spbt_q, _, wsem_q = bq
        pltpu.make_async_copy(eo_lin.at[0, :, pl.ds(0, _CHUNK)], acct_q,
                              wsem_q).wait()
        pltpu.make_async_copy(eo_lin.at[0, :, pl.ds(0, _CHUNK)], spbt_q,
                              wsem_q).wait()

    def segment(g, p):
        bp, bq = bufs[p], bufs[1 - p]
        _, rows_p, acct_p, sidx_p, srow_p, spbt_p, _, wsem_p = bp
        drain_gathers(bp)

        @pl.when(g < nchunks - 1)
        def _():
            @pl.when(g >= 1)
            def _():
                drain_writes(bq)
            stage_and_fire(g + 1, bq)

        for k in range(_CHUNK // 16):
            s = sidx_p[pl.ds(k * 16, 16)]
            sp = jnp.where(s == 0, 1, s)
            sp = jnp.where(sp > 1, sp - 1, sp)
            sp = jnp.minimum(sp, MHD)
            r_v[pl.ds(k * 16, 16)] = 1.0 / sp.astype(jnp.float32)

        def acc_body(pp, c2):
            a0 = rows_p[pp, pl.ds(0, 16)]
            a1 = rows_p[pp, pl.ds(16, 16)]
            for m in range(1, NSLAB):
                a0 = a0 + rows_p[m * _CHUNK + pp, pl.ds(0, 16)]
                a1 = a1 + rows_p[m * _CHUNK + pp, pl.ds(16, 16)]
            col = jnp.full((16,), pp, dtype=jnp.int32)
            rs = plsc.load_gather(r_v, [col])
            plsc.store_scatter(acct_p, [lanes, col], a0 * rs)
            plsc.store_scatter(acct_p, [lanes + 16, col], a1 * rs)
            s0 = srow_p[pp, pl.ds(0, 16)]
            s1 = srow_p[pp, pl.ds(16, 16)]
            plsc.store_scatter(spbt_p, [lanes, col], s0)
            plsc.store_scatter(spbt_p, [lanes + 16, col], s1)
            return c2

        lax.fori_loop(0, _CHUNK, acc_body, 0)

        pij0 = pl.multiple_of(g * _CHUNK, _CHUNK)
        pltpu.async_copy(acct_p, eo_lin.at[w, :, pl.ds(pij0, _CHUNK)], wsem_p)
        pltpu.async_copy(spbt_p, spb_lin.at[w, :, pl.ds(pij0, _CHUNK)], wsem_p)

    stage_and_fire(0, bufs[0])

    def body(i, carry):
        segment(2 * i, 0)
        segment(2 * i + 1, 1)
        return carry

    lax.fori_loop(0, nchunks // 2, body, 0)
    drain_writes(bufs[0])
    drain_writes(bufs[1])


def _sc_gather(tbls5, cidx3, spos_flat, spatial_w, n_graph, pn):
    nchunks = pn // _CHUNK
    mesh = plsc.VectorSubcoreMesh(core_axis_name="c", subcore_axis_name="s")
    dbl = lambda t: [t, t]
    fn = pl.kernel(
        functools.partial(_sc_body, nchunks, pn),
        mesh=mesh,
        out_type=[
            jax.ShapeDtypeStruct((n_graph, H, pn), jnp.float32),
            jax.ShapeDtypeStruct((n_graph, H, pn), jnp.float32),
        ],
        scratch_types=(
            dbl(pltpu.VMEM((NSLAB, _CHUNK), jnp.int32))
            + dbl(pltpu.VMEM((NSLAB * _CHUNK, H), jnp.float32))
            + dbl(pltpu.VMEM((H, _CHUNK), jnp.float32))
            + dbl(pltpu.VMEM((_CHUNK,), jnp.int32))
            + dbl(pltpu.VMEM((_CHUNK, H), jnp.float32))
            + dbl(pltpu.VMEM((H, _CHUNK), jnp.float32))
            + [pltpu.VMEM((_CHUNK,), jnp.float32)]
            + [pltpu.SemaphoreType.DMA] * 4
        ),
        compiler_params=pltpu.CompilerParams(use_tc_tiling_on_sc=False,
                                             needs_layout_passes=False),
    )
    return fn(*tbls5, cidx3, spos_flat, spatial_w)


def _gab_body(ab_ref, vd_ref, gab_ref):
    ab = ab_ref[0]
    ri = lax.broadcasted_iota(jnp.int32, ab.shape, 0)
    ci = lax.broadcasted_iota(jnp.int32, ab.shape, 1)
    mask = jnp.where((ri == 0) | (ci == 0), 1.0, 0.0)
    vd = vd_ref[...]
    gab_ref[0] = (2.0 * ab[:, None, :] + vd[0][None, :, None] * mask[:, None, :])


def _gab(attn_bias, vdist_w, n_graph):
    np1 = attn_bias.shape[1]
    return pl.pallas_call(
        _gab_body,
        grid=(n_graph,),
        in_specs=[
            pl.BlockSpec((1, np1, np1), lambda b: (b, 0, 0)),
            pl.BlockSpec((1, H), lambda b: (0, 0)),
        ],
        out_specs=pl.BlockSpec((1, np1, H, np1), lambda b: (b, 0, 0, 0)),
        out_shape=jax.ShapeDtypeStruct((n_graph, np1, H, np1), jnp.float32),
    )(attn_bias, vdist_w)


def kernel(attn_bias, spatial_pos, x, edge_input, attn_edge_type,
           edge_enc_w, edge_dis_w, spatial_w, vdist_w):
    n_graph, n_node = x.shape[0], x.shape[1]
    pn = n_node * n_node
    npos = n_graph * pn

    w5 = edge_dis_w.reshape(-1, H, H)[:MHD]
    tstack = _build_table(edge_enc_w.astype(jnp.float32), w5)
    tbls5 = [tstack[d] for d in range(MHD)]

    cidx3 = (edge_input.astype(jnp.int32)
             .transpose(0, 3, 4, 1, 2).reshape(n_graph, NSLAB, pn))
    spos_flat = spatial_pos.astype(jnp.int32).reshape(npos)

    eo3, spb3 = _sc_gather(tbls5, cidx3, spos_flat,
                           spatial_w.astype(jnp.float32), n_graph, pn)

    gab_p = _gab(attn_bias, vdist_w, n_graph)
    gab = gab_p.transpose(0, 2, 1, 3)
    spb = spb3.reshape(n_graph, H, n_node, n_node)
    eo = eo3.reshape(n_graph, H, n_node, n_node)
    return (gab, spb, spb, eo, eo)


# speedup vs baseline: 1.0214x; 1.0214x over previous
import functools

import jax
import jax.numpy as jnp
from jax import lax
from jax.experimental import pallas as pl
from jax.experimental.pallas import tpu as pltpu
from jax.experimental.pallas import tpu_sc as plsc

H = 32
NE = 1536
NS = 512
MHD = 5
NSLAB = MHD * 3

_NC = 2
_NSUB = 16
_NW = _NC * _NSUB

_CHUNK = 64


def _tmul_body(ew_ref, w_ref, out_ref):
    out_ref[0] = jnp.dot(ew_ref[...], w_ref[0],
                         preferred_element_type=jnp.float32) * (1.0 / 3.0)


def _build_table(edge_enc_w, w5):
    return pl.pallas_call(
        _tmul_body,
        grid=(MHD,),
        in_specs=[
            pl.BlockSpec((NE + 1, H), lambda d: (0, 0)),
            pl.BlockSpec((1, H, H), lambda d: (d, 0, 0)),
        ],
        out_specs=pl.BlockSpec((1, NE + 1, H), lambda d: (d, 0, 0)),
        out_shape=jax.ShapeDtypeStruct((MHD, NE + 1, H), jnp.float32),
    )(edge_enc_w, w5)


def _sc_body(nchunks, pn, t0, t1, t2, t3, t4, cidx, spos, spw, eo_lin, spb_lin,
             idx0, idx1, rows0, rows1, acct0, acct1, sidx0, sidx1,
             srow0, srow1, spbt0, spbt1, acc0, acc1, r_v,
             gsem0, gsem1, wsem0, wsem1):
    tbls = (t0, t1, t2, t3, t4)
    w = lax.axis_index("s") * _NC + lax.axis_index("c")
    base = w * pn
    lanes = jnp.arange(16, dtype=jnp.int32)
    bufs = ((idx0, rows0, acct0, sidx0, srow0, spbt0, acc0, gsem0, wsem0),
            (idx1, rows1, acct1, sidx1, srow1, spbt1, acc1, gsem1, wsem1))

    def stage_and_fire(g, bq):
        idx_q, rows_q, _, sidx_q, srow_q, _, _, gsem_q, _ = bq
        pij0 = pl.multiple_of(g * _CHUNK, _CHUNK)
        pos0 = pl.multiple_of(base + pij0, _CHUNK)
        pltpu.sync_copy(cidx.at[w, :, pl.ds(pij0, _CHUNK)], idx_q)
        pltpu.sync_copy(spos.at[pl.ds(pos0, _CHUNK)], sidx_q)
        for m in range(NSLAB):
            pltpu.async_copy(tbls[m // 3].at[idx_q.at[m]],
                             rows_q.at[pl.ds(m * _CHUNK, _CHUNK)], gsem_q)
        pltpu.async_copy(spw.at[sidx_q], srow_q, gsem_q)

    def drain_gathers(bp):
        _, rows_p, _, _, srow_p, _, _, gsem_p, _ = bp
        pltpu.make_async_copy(t0.at[pl.ds(0, NSLAB * _CHUNK)], rows_p,
                              gsem_p).wait()
        pltpu.make_async_copy(t0.at[pl.ds(0, _CHUNK)], srow_p, gsem_p).wait()

    def drain_writes(bq):
        _, _, acct_q, _, _, spbt_q, _, _, wsem_q = bq
        pltpu.make_async_copy(eo_lin.at[0, :, pl.ds(0, _CHUNK)], acct_q,
                              wsem_q).wait()
        pltpu.make_async_copy(eo_lin.at[0, :, pl.ds(0, _CHUNK)], spbt_q,
                              wsem_q).wait()

    def segment(g, p):
        bp, bq = bufs[p], bufs[1 - p]
        _, rows_p, acct_p, sidx_p, srow_p, spbt_p, acc_p, _, wsem_p = bp
        drain_gathers(bp)

        @pl.when(g < nchunks - 1)
        def _():
            @pl.when(g >= 1)
            def _():
                drain_writes(bq)
            stage_and_fire(g + 1, bq)

        for k in range(_CHUNK // 16):
            s = sidx_p[pl.ds(k * 16, 16)]
            sp = jnp.where(s == 0, 1, s)
            sp = jnp.where(sp > 1, sp - 1, sp)
            sp = jnp.minimum(sp, MHD)
            r_v[pl.ds(k * 16, 16)] = 1.0 / sp.astype(jnp.float32)

        def acc_body(pp, c2):
            a0 = rows_p[pp, pl.ds(0, 16)]
            a1 = rows_p[pp, pl.ds(16, 16)]
            for m in range(1, NSLAB):
                a0 = a0 + rows_p[m * _CHUNK + pp, pl.ds(0, 16)]
                a1 = a1 + rows_p[m * _CHUNK + pp, pl.ds(16, 16)]
            acc_p[pp, pl.ds(0, 16)] = a0
            acc_p[pp, pl.ds(16, 16)] = a1
            return c2

        lax.fori_loop(0, _CHUNK, acc_body, 0)

        for sub in range(_CHUNK // 16):
            rows16 = lanes + sub * 16
            rsub = r_v[pl.ds(sub * 16, 16)]
            for h in range(H):
                colh = jnp.full((16,), h, dtype=jnp.int32)
                acct_p[h, pl.ds(sub * 16, 16)] = (
                    plsc.load_gather(acc_p, [rows16, colh]) * rsub)
                spbt_p[h, pl.ds(sub * 16, 16)] = (
                    plsc.load_gather(srow_p, [rows16, colh]))

        pij0 = pl.multiple_of(g * _CHUNK, _CHUNK)
        pltpu.async_copy(acct_p, eo_lin.at[w, :, pl.ds(pij0, _CHUNK)], wsem_p)
        pltpu.async_copy(spbt_p, spb_lin.at[w, :, pl.ds(pij0, _CHUNK)], wsem_p)

    stage_and_fire(0, bufs[0])

    def body(i, carry):
        segment(2 * i, 0)
        segment(2 * i + 1, 1)
        return carry

    lax.fori_loop(0, nchunks // 2, body, 0)
    drain_writes(bufs[0])
    drain_writes(bufs[1])


def _sc_gather(tbls5, cidx3, spos_flat, spatial_w, n_graph, pn):
    nchunks = pn // _CHUNK
    mesh = plsc.VectorSubcoreMesh(core_axis_name="c", subcore_axis_name="s")
    dbl = lambda t: [t, t]
    fn = pl.kernel(
        functools.partial(_sc_body, nchunks, pn),
        mesh=mesh,
        out_type=[
            jax.ShapeDtypeStruct((n_graph, H, pn), jnp.float32),
            jax.ShapeDtypeStruct((n_graph, H, pn), jnp.float32),
        ],
        scratch_types=(
            dbl(pltpu.VMEM((NSLAB, _CHUNK), jnp.int32))
            + dbl(pltpu.VMEM((NSLAB * _CHUNK, H), jnp.float32))
            + dbl(pltpu.VMEM((H, _CHUNK), jnp.float32))
            + dbl(pltpu.VMEM((_CHUNK,), jnp.int32))
            + dbl(pltpu.VMEM((_CHUNK, H), jnp.float32))
            + dbl(pltpu.VMEM((H, _CHUNK), jnp.float32))
            + dbl(pltpu.VMEM((_CHUNK, H), jnp.float32))
            + [pltpu.VMEM((_CHUNK,), jnp.float32)]
            + [pltpu.SemaphoreType.DMA] * 4
        ),
        compiler_params=pltpu.CompilerParams(use_tc_tiling_on_sc=False,
                                             needs_layout_passes=False),
    )
    return fn(*tbls5, cidx3, spos_flat, spatial_w)


def _gab_body(ab_ref, vd_ref, gab_ref):
    ab = ab_ref[0]
    ri = lax.broadcasted_iota(jnp.int32, ab.shape, 0)
    ci = lax.broadcasted_iota(jnp.int32, ab.shape, 1)
    mask = jnp.where((ri == 0) | (ci == 0), 1.0, 0.0)
    vd = vd_ref[...]
    gab_ref[0] = (2.0 * ab[:, None, :] + vd[0][None, :, None] * mask[:, None, :])


def _gab(attn_bias, vdist_w, n_graph):
    np1 = attn_bias.shape[1]
    return pl.pallas_call(
        _gab_body,
        grid=(n_graph,),
        in_specs=[
            pl.BlockSpec((1, np1, np1), lambda b: (b, 0, 0)),
            pl.BlockSpec((1, H), lambda b: (0, 0)),
        ],
        out_specs=pl.BlockSpec((1, np1, H, np1), lambda b: (b, 0, 0, 0)),
        out_shape=jax.ShapeDtypeStruct((n_graph, np1, H, np1), jnp.float32),
    )(attn_bias, vdist_w)


def kernel(attn_bias, spatial_pos, x, edge_input, attn_edge_type,
           edge_enc_w, edge_dis_w, spatial_w, vdist_w):
    n_graph, n_node = x.shape[0], x.shape[1]
    pn = n_node * n_node
    npos = n_graph * pn

    w5 = edge_dis_w.reshape(-1, H, H)[:MHD]
    tstack = _build_table(edge_enc_w.astype(jnp.float32), w5)
    tbls5 = [tstack[d] for d in range(MHD)]

    cidx3 = (edge_input.astype(jnp.int32)
             .transpose(0, 3, 4, 1, 2).reshape(n_graph, NSLAB, pn))
    spos_flat = spatial_pos.astype(jnp.int32).reshape(npos)

    eo3, spb3 = _sc_gather(tbls5, cidx3, spos_flat,
                           spatial_w.astype(jnp.float32), n_graph, pn)

    gab_p = _gab(attn_bias, vdist_w, n_graph)
    gab = gab_p.transpose(0, 2, 1, 3)
    spb = spb3.reshape(n_graph, H, n_node, n_node)
    eo = eo3.reshape(n_graph, H, n_node, n_node)
    return (gab, spb, spb, eo, eo)
